# baseline (speedup 1.0000x reference)
import functools

import jax
import jax.numpy as jnp
from jax import lax
from jax.experimental import pallas as pl
from jax.experimental.pallas import tpu as pltpu
from jax.experimental.pallas import tpu_sc as plsc


def _mm_body(x_ref, w_ref, o_ref):
    r = jnp.maximum(
        jnp.dot(x_ref[...], w_ref[...], preferred_element_type=jnp.float32), 0.0)
    dh = r.shape[1] // 2
    o_ref[0] = r[:, :dh]
    o_ref[1] = r[:, dh:]


def _relu_matmul_split(x, W):
    N, Din = x.shape
    Dout = W.shape[1]
    Dh = Dout // 2
    BN = 1000
    grid = (N // BN,)
    return pl.pallas_call(
        _mm_body,
        grid=grid,
        in_specs=[
            pl.BlockSpec((BN, Din), lambda i: (i, 0)),
            pl.BlockSpec((Din, Dout), lambda i: (0, 0)),
        ],
        out_specs=pl.BlockSpec((2, BN, Dh), lambda i: (0, i, 0)),
        out_shape=jax.ShapeDtypeStruct((2, N, Dh), jnp.float32),
    )(x, W)


def _make_sc_aggregate(N, E, Dh, Npad, C, rpt):
    info = plsc.get_sparse_core_info()
    NC, NS = info.num_cores, info.num_subcores
    nch = E // C
    base_ch = nch // NS
    extra = nch - base_ch * NS
    GK = 6
    ngr = base_ch // GK
    assert ngr * GK == base_ch
    ZR = rpt // 8

    mesh = plsc.VectorSubcoreMesh(core_axis_name="c", subcore_axis_name="s")

    @functools.partial(
        pl.kernel,
        out_type=[
            jax.ShapeDtypeStruct((NC, Npad, Dh), jnp.float32),
            jax.ShapeDtypeStruct((NC, Npad, 16), jnp.float32),
        ],
        mesh=mesh,
        compiler_params=pltpu.CompilerParams(use_tc_tiling_on_sc=False),
        scratch_types=[
            pltpu.VMEM_SHARED((Npad, Dh), jnp.float32),
            pltpu.VMEM_SHARED((Npad, 16), jnp.float32),
            pltpu.VMEM((3, GK * C), jnp.int32),
            pltpu.VMEM((3, GK * C), jnp.int32),
            pltpu.VMEM((4, C), jnp.int32),
            pltpu.VMEM((4, C), jnp.int32),
            pltpu.VMEM((4, C, Dh), jnp.float32),
            pltpu.VMEM((C, 16), jnp.float32),
            pltpu.VMEM((ZR, 16), jnp.float32),
            pltpu.SemaphoreType.DMA((4,)),
            pltpu.SemaphoreType.DMA((3,)),
            pltpu.SemaphoreType.DMA((4,)),
        ],
    )
    def sc_agg(feat_hbm, edge_hbm, sum_hbm, cnt_hbm,
               acc, cnt, grow_v, gcol_v, rowm_v, colm_v, rows_v, half_v, zc_v,
               sem, gsem, ssem):
        c = lax.axis_index("c")
        s = lax.axis_index("s")

        zero16 = jnp.zeros((16,), jnp.float32)
        half16 = jnp.full((16,), 0.5, jnp.float32)

        def fill_half(i, _):
            half_v[i, :] = half16
            return 0
        lax.fori_loop(0, C, fill_half, 0)

        def fill_zc(i, _):
            zc_v[i, :] = zero16
            return 0
        lax.fori_loop(0, ZR, fill_zc, 0)

        def fill_z(i, _):
            rows_v[0, i // (Dh // 16), pl.ds((i % (Dh // 16)) * 16, 16)] = zero16
            return 0
        lax.fori_loop(0, C * (Dh // 16), fill_z, 0)

        rbase = s * rpt
        nz = rpt // C
        rem_rows = rpt - nz * C
        for k in range(nz):
            pltpu.sync_copy(rows_v.at[0], acc.at[pl.ds(rbase + k * C, C)])
        if rem_rows:
            pltpu.sync_copy(rows_v.at[0, pl.ds(0, rem_rows)],
                            acc.at[pl.ds(rbase + nz * C, rem_rows)])
        for k in range(8):
            pltpu.sync_copy(zc_v, cnt.at[pl.ds(rbase + k * ZR, ZR)])
        plsc.subcore_barrier()

        cstart = base_ch * s
        coffs = c * N

        def load_group(g):
            gb = lax.rem(g, 3)
            off = (cstart + g * GK) * C
            pltpu.async_copy(edge_hbm.at[pl.ds(off, GK * C)],
                             grow_v.at[gb], gsem.at[gb])
            pltpu.async_copy(edge_hbm.at[pl.ds(E + off, GK * C)],
                             gcol_v.at[gb], gsem.at[gb])

        def wait_group(g):
            gb = lax.rem(g, 3)
            pltpu.make_async_copy(edge_hbm.at[pl.ds(0, GK * C)],
                                  grow_v.at[gb], gsem.at[gb]).wait()
            pltpu.make_async_copy(edge_hbm.at[pl.ds(0, GK * C)],
                                  gcol_v.at[gb], gsem.at[gb]).wait()

        def stage(i):
            b = lax.rem(i, 4)
            gb = lax.rem(i // GK, 3)
            kofs = lax.rem(i, GK) * C
            for j in range(C // 16):
                r = grow_v[gb, pl.ds(kofs + j * 16, 16)]
                cc = gcol_v[gb, pl.ds(kofs + j * 16, 16)]
                rowm_v[b, pl.ds(j * 16, 16)] = jnp.where(r == cc, N, r)
                colm_v[b, pl.ds(j * 16, 16)] = cc + coffs
            pltpu.async_copy(feat_hbm.at[colm_v.at[b]], rows_v.at[b], sem.at[b])

        def wait_gather(i):
            b = lax.rem(i, 4)
            pltpu.make_async_copy(
                feat_hbm.at[colm_v.at[b]], rows_v.at[b], sem.at[b]).wait()

        def start_scatters(i):
            b = lax.rem(i, 4)
            pltpu.async_copy(rows_v.at[b], acc.at[rowm_v.at[b]],
                             ssem.at[b], add=True)
            pltpu.async_copy(half_v, cnt.at[rowm_v.at[b]],
                             ssem.at[b], add=True)

        def wait_scatters(i):
            b = lax.rem(i, 4)
            pltpu.make_async_copy(rows_v.at[b], acc.at[pl.ds(0, C)],
                                  ssem.at[b]).wait()
            pltpu.make_async_copy(half_v, cnt.at[pl.ds(0, C)],
                                  ssem.at[b]).wait()

        ntot = base_ch
        load_group(0)
        wait_group(0)
        load_group(1)
        stage(0)
        stage(1)

        def body(i, _):
            g2 = (i + 2) // GK
            at_g2 = jnp.logical_and(lax.rem(i + 2, GK) == 0, i + 2 < ntot)

            @pl.when(i > 1)
            def _():
                wait_scatters(i - 2)

            @pl.when(at_g2)
            def _():
                wait_group(g2)

            @pl.when(jnp.logical_and(at_g2, g2 + 1 < ngr))
            def _():
                load_group(g2 + 1)

            @pl.when(i + 2 < ntot)
            def _():
                stage(i + 2)

            wait_gather(i)
            start_scatters(i)
            return 0
        lax.fori_loop(0, ntot, body, 0)
        wait_scatters(ntot - 2)
        wait_scatters(ntot - 1)

        @pl.when(s < extra)
        def _():
            toff = (NS * base_ch + s) * C
            pltpu.sync_copy(edge_hbm.at[pl.ds(toff, C)],
                            grow_v.at[0, pl.ds(0, C)])
            pltpu.sync_copy(edge_hbm.at[pl.ds(E + toff, C)],
                            gcol_v.at[0, pl.ds(0, C)])
            for j in range(C // 16):
                r = grow_v[0, pl.ds(j * 16, 16)]
                cc = gcol_v[0, pl.ds(j * 16, 16)]
                rowm_v[0, pl.ds(j * 16, 16)] = jnp.where(r == cc, N, r)
                colm_v[0, pl.ds(j * 16, 16)] = cc + coffs
            pltpu.async_copy(feat_hbm.at[colm_v.at[0]], rows_v.at[0], sem.at[0])
            pltpu.make_async_copy(
                feat_hbm.at[colm_v.at[0]], rows_v.at[0], sem.at[0]).wait()
            pltpu.sync_copy(rows_v.at[0], acc.at[rowm_v.at[0]], add=True)
            pltpu.sync_copy(half_v, cnt.at[rowm_v.at[0]], add=True)

        plsc.subcore_barrier()
        pltpu.sync_copy(acc.at[pl.ds(rbase, rpt)], sum_hbm.at[c, pl.ds(rbase, rpt)])
        pltpu.sync_copy(cnt.at[pl.ds(rbase, rpt)], cnt_hbm.at[c, pl.ds(rbase, rpt)])

    return sc_agg


def _div_body(ps_ref, pc_ref, feat_ref, o_ref):
    lo = ps_ref[0] + feat_ref[0]
    hi = ps_ref[1] + feat_ref[1]
    den = pc_ref[0, :, 0:1] + pc_ref[1, :, 0:1] + 1.0
    o_ref[...] = jnp.concatenate([lo, hi], axis=1) / den


def _combine(psum, pcnt, feat2, N, D):
    BN = 400
    Dh = D // 2
    grid = (N // BN,)
    return pl.pallas_call(
        _div_body,
        grid=grid,
        in_specs=[
            pl.BlockSpec((2, BN, Dh), lambda i: (0, i, 0)),
            pl.BlockSpec((2, BN, 16), lambda i: (0, i, 0)),
            pl.BlockSpec((2, BN, Dh), lambda i: (0, i, 0)),
        ],
        out_specs=pl.BlockSpec((BN, D), lambda i: (i, 0)),
        out_shape=jax.ShapeDtypeStruct((N, D), jnp.float32),
    )(psum, pcnt, feat2)


def kernel(x, edge_index, W):
    N, _ = x.shape
    D = W.shape[1]
    E = edge_index.shape[1]
    Dh = D // 2

    info = plsc.get_sparse_core_info()
    NS = info.num_subcores
    C = 128
    assert E % C == 0
    rpt = -(-(N + 1) // NS)
    rpt = -(-rpt // 8) * 8
    Npad = rpt * NS

    feat2 = _relu_matmul_split(x, W)
    psum, pcnt = _make_sc_aggregate(N, E, Dh, Npad, C, rpt)(
        feat2.reshape(2 * N, Dh), edge_index.reshape(-1))
    return _combine(psum, pcnt, feat2, N, D)


# speedup vs baseline: 1.0726x; 1.0726x over previous
import functools

import jax
import jax.numpy as jnp
from jax import lax
from jax.experimental import pallas as pl
from jax.experimental.pallas import tpu as pltpu
from jax.experimental.pallas import tpu_sc as plsc


def _mm_body(x_ref, w_ref, o_ref):
    o_ref[...] = jnp.maximum(
        jnp.dot(x_ref[...], w_ref[...], preferred_element_type=jnp.float32), 0.0)


def _relu_matmul(x, W):
    N, Din = x.shape
    Dout = W.shape[1]
    BN = 1000
    grid = (N // BN,)
    return pl.pallas_call(
        _mm_body,
        grid=grid,
        in_specs=[
            pl.BlockSpec((BN, Din), lambda i: (i, 0)),
            pl.BlockSpec((Din, Dout), lambda i: (0, 0)),
        ],
        out_specs=pl.BlockSpec((BN, Dout), lambda i: (i, 0)),
        out_shape=jax.ShapeDtypeStruct((N, Dout), jnp.float32),
    )(x, W)


def _make_sc_aggregate(N, E, D, Npad, C, rpt):
    info = plsc.get_sparse_core_info()
    NC, NS = info.num_cores, info.num_subcores
    NW = NC * NS
    nch = E // C
    base_ch = nch // NW
    extra = nch - base_ch * NW
    GK = 6
    ngr = base_ch // GK
    assert ngr * GK == base_ch
    ZR = rpt // 8

    mesh = plsc.VectorSubcoreMesh(core_axis_name="c", subcore_axis_name="s")

    @functools.partial(
        pl.kernel,
        out_type=[
            jax.ShapeDtypeStruct((NC, Npad, D), jnp.float32),
            jax.ShapeDtypeStruct((NC, Npad, 16), jnp.float32),
        ],
        mesh=mesh,
        compiler_params=pltpu.CompilerParams(use_tc_tiling_on_sc=False),
        scratch_types=[
            pltpu.VMEM_SHARED((Npad, D), jnp.float32),
            pltpu.VMEM_SHARED((Npad, 16), jnp.float32),
            pltpu.VMEM((2, GK * C), jnp.int32),
            pltpu.VMEM((2, GK * C), jnp.int32),
            pltpu.VMEM((2, C), jnp.int32),
            pltpu.VMEM((2, C, D), jnp.float32),
            pltpu.VMEM((C, 16), jnp.float32),
            pltpu.VMEM((ZR, 16), jnp.float32),
            pltpu.SemaphoreType.DMA((2,)),
            pltpu.SemaphoreType.DMA((2,)),
            pltpu.SemaphoreType.DMA((2,)),
        ],
    )
    def sc_agg(feat_hbm, edge_hbm, sum_hbm, cnt_hbm,
               acc, cnt, grow_v, gcol_v, rowm_v, rows_v, ones_v, zc_v,
               sem, gsem, ssem):
        c = lax.axis_index("c")
        s = lax.axis_index("s")
        wid = c * NS + s
        cstart = base_ch * wid

        def load_group(g):
            gb = lax.rem(g, 2)
            off = (cstart + g * GK) * C
            pltpu.async_copy(edge_hbm.at[pl.ds(off, GK * C)],
                             grow_v.at[gb], gsem.at[gb])
            pltpu.async_copy(edge_hbm.at[pl.ds(E + off, GK * C)],
                             gcol_v.at[gb], gsem.at[gb])

        load_group(0)

        zero16 = jnp.zeros((16,), jnp.float32)
        one16 = jnp.ones((16,), jnp.float32)

        def fill_ones(i, _):
            ones_v[i, :] = one16
            return 0
        lax.fori_loop(0, C, fill_ones, 0)

        def fill_zc(i, _):
            zc_v[i, :] = zero16
            return 0
        lax.fori_loop(0, ZR, fill_zc, 0)

        def fill_z(i, _):
            rows_v[0, i // (D // 16), pl.ds((i % (D // 16)) * 16, 16)] = zero16
            return 0
        lax.fori_loop(0, C * (D // 16), fill_z, 0)

        rbase = s * rpt
        nz = rpt // C
        rem_rows = rpt - nz * C
        for k in range(nz):
            pltpu.async_copy(rows_v.at[0], acc.at[pl.ds(rbase + k * C, C)],
                             ssem.at[0])
        if rem_rows:
            pltpu.async_copy(rows_v.at[0, pl.ds(0, rem_rows)],
                             acc.at[pl.ds(rbase + nz * C, rem_rows)], ssem.at[0])
        for k in range(8):
            pltpu.async_copy(zc_v, cnt.at[pl.ds(rbase + k * ZR, ZR)], ssem.at[1])
        for k in range(nz):
            pltpu.make_async_copy(rows_v.at[0], acc.at[pl.ds(rbase + k * C, C)],
                                  ssem.at[0]).wait()
        if rem_rows:
            pltpu.make_async_copy(rows_v.at[0, pl.ds(0, rem_rows)],
                                  acc.at[pl.ds(rbase + nz * C, rem_rows)],
                                  ssem.at[0]).wait()
        for k in range(8):
            pltpu.make_async_copy(zc_v, cnt.at[pl.ds(rbase + k * ZR, ZR)],
                                  ssem.at[1]).wait()
        plsc.subcore_barrier()

        def wait_group(g):
            gb = lax.rem(g, 2)
            pltpu.make_async_copy(edge_hbm.at[pl.ds(0, GK * C)],
                                  grow_v.at[gb], gsem.at[gb]).wait()
            pltpu.make_async_copy(edge_hbm.at[pl.ds(0, GK * C)],
                                  gcol_v.at[gb], gsem.at[gb]).wait()

        def stage(i):
            b = lax.rem(i, 2)
            g = i // GK
            gb = lax.rem(g, 2)
            kofs = lax.rem(i, GK) * C
            for j in range(C // 16):
                r = grow_v[gb, pl.ds(kofs + j * 16, 16)]
                cc = gcol_v[gb, pl.ds(kofs + j * 16, 16)]
                rowm_v[b, pl.ds(j * 16, 16)] = jnp.where(r == cc, N, r)
            pltpu.async_copy(feat_hbm.at[gcol_v.at[gb, pl.ds(kofs, C)]],
                             rows_v.at[b], sem.at[b])

        def wait_gather(i):
            b = lax.rem(i, 2)
            g = i // GK
            gb = lax.rem(g, 2)
            kofs = lax.rem(i, GK) * C
            pltpu.make_async_copy(
                feat_hbm.at[gcol_v.at[gb, pl.ds(kofs, C)]],
                rows_v.at[b], sem.at[b]).wait()

        def start_scatters(i):
            b = lax.rem(i, 2)
            pltpu.async_copy(rows_v.at[b], acc.at[rowm_v.at[b]],
                             ssem.at[b], add=True)
            pltpu.async_copy(ones_v, cnt.at[rowm_v.at[b]],
                             ssem.at[b], add=True)

        def wait_scatters(i):
            b = lax.rem(i, 2)
            pltpu.make_async_copy(rows_v.at[b], acc.at[pl.ds(0, C)],
                                  ssem.at[b]).wait()
            pltpu.make_async_copy(ones_v, cnt.at[pl.ds(0, C)],
                                  ssem.at[b]).wait()

        ntot = ngr * GK

        def body(i, _):
            g = i // GK
            at_group = lax.rem(i, GK) == 0

            @pl.when(i > 1)
            def _():
                wait_scatters(i - 2)

            @pl.when(jnp.logical_and(at_group, i > 0))
            def _():
                wait_gather(i - 1)

            @pl.when(at_group)
            def _():
                wait_group(g)

            @pl.when(jnp.logical_and(at_group, g + 1 < ngr))
            def _():
                load_group(g + 1)

            stage(i)

            @pl.when(jnp.logical_and(jnp.logical_not(at_group), i > 0))
            def _():
                wait_gather(i - 1)

            @pl.when(i > 0)
            def _():
                start_scatters(i - 1)
            return 0
        lax.fori_loop(0, ntot, body, 0)
        wait_gather(ntot - 1)
        start_scatters(ntot - 1)
        wait_scatters(ntot - 2)
        wait_scatters(ntot - 1)

        @pl.when(wid < extra)
        def _():
            toff = (NW * base_ch + wid) * C
            pltpu.sync_copy(edge_hbm.at[pl.ds(toff, C)],
                            grow_v.at[0, pl.ds(0, C)])
            pltpu.sync_copy(edge_hbm.at[pl.ds(E + toff, C)],
                            gcol_v.at[0, pl.ds(0, C)])
            for j in range(C // 16):
                r = grow_v[0, pl.ds(j * 16, 16)]
                cc = gcol_v[0, pl.ds(j * 16, 16)]
                rowm_v[0, pl.ds(j * 16, 16)] = jnp.where(r == cc, N, r)
            pltpu.async_copy(feat_hbm.at[gcol_v.at[0, pl.ds(0, C)]],
                             rows_v.at[0], sem.at[0])
            pltpu.make_async_copy(
                feat_hbm.at[gcol_v.at[0, pl.ds(0, C)]],
                rows_v.at[0], sem.at[0]).wait()
            pltpu.sync_copy(rows_v.at[0], acc.at[rowm_v.at[0]], add=True)
            pltpu.sync_copy(ones_v, cnt.at[rowm_v.at[0]], add=True)

        plsc.subcore_barrier()
        pltpu.async_copy(acc.at[pl.ds(rbase, rpt)],
                         sum_hbm.at[c, pl.ds(rbase, rpt)], sem.at[0])
        pltpu.async_copy(cnt.at[pl.ds(rbase, rpt)],
                         cnt_hbm.at[c, pl.ds(rbase, rpt)], sem.at[1])
        pltpu.make_async_copy(acc.at[pl.ds(rbase, rpt)],
                              sum_hbm.at[c, pl.ds(rbase, rpt)], sem.at[0]).wait()
        pltpu.make_async_copy(cnt.at[pl.ds(rbase, rpt)],
                              cnt_hbm.at[c, pl.ds(rbase, rpt)], sem.at[1]).wait()

    return sc_agg


def _div_body(ps_ref, pc_ref, feat_ref, o_ref):
    total = ps_ref[0] + ps_ref[1] + feat_ref[...]
    den = pc_ref[0, :, 0:1] + pc_ref[1, :, 0:1] + 1.0
    o_ref[...] = total / den


def _combine(psum, pcnt, feat):
    N, D = feat.shape
    BN = 400
    grid = (N // BN,)
    return pl.pallas_call(
        _div_body,
        grid=grid,
        in_specs=[
            pl.BlockSpec((2, BN, D), lambda i: (0, i, 0)),
            pl.BlockSpec((2, BN, 16), lambda i: (0, i, 0)),
            pl.BlockSpec((BN, D), lambda i: (i, 0)),
        ],
        out_specs=pl.BlockSpec((BN, D), lambda i: (i, 0)),
        out_shape=jax.ShapeDtypeStruct((N, D), jnp.float32),
    )(psum, pcnt, feat)


def kernel(x, edge_index, W):
    N, _ = x.shape
    D = W.shape[1]
    E = edge_index.shape[1]

    info = plsc.get_sparse_core_info()
    NS = info.num_subcores
    C = 128
    assert E % C == 0
    rpt = -(-(N + 1) // NS)
    rpt = -(-rpt // 8) * 8
    Npad = rpt * NS

    feat = _relu_matmul(x, W)
    psum, pcnt = _make_sc_aggregate(N, E, D, Npad, C, rpt)(
        feat, edge_index.reshape(-1))
    return _combine(psum, pcnt, feat)


# speedup vs baseline: 1.0927x; 1.0187x over previous
import functools

import jax
import jax.numpy as jnp
from jax import lax
from jax.experimental import pallas as pl
from jax.experimental.pallas import tpu as pltpu
from jax.experimental.pallas import tpu_sc as plsc


def _mm_body(x_ref, w_ref, o_ref):
    o_ref[...] = jnp.maximum(
        jnp.dot(x_ref[...], w_ref[...], preferred_element_type=jnp.float32), 0.0)


def _relu_matmul(x, W):
    N, Din = x.shape
    Dout = W.shape[1]
    BN = 1000
    grid = (N // BN,)
    return pl.pallas_call(
        _mm_body,
        grid=grid,
        in_specs=[
            pl.BlockSpec((BN, Din), lambda i: (i, 0)),
            pl.BlockSpec((Din, Dout), lambda i: (0, 0)),
        ],
        out_specs=pl.BlockSpec((BN, Dout), lambda i: (i, 0)),
        out_shape=jax.ShapeDtypeStruct((N, Dout), jnp.float32),
    )(x, W)


def _make_sc_aggregate(N, E, D, Npad, C, rpt):
    info = plsc.get_sparse_core_info()
    NC, NS = info.num_cores, info.num_subcores
    NW = NC * NS
    nch = E // C
    base_ch = nch // NW
    extra = nch - base_ch * NW
    GK = 6
    ngr = base_ch // GK
    assert ngr * GK == base_ch
    ZR = rpt // 8

    mesh = plsc.VectorSubcoreMesh(core_axis_name="c", subcore_axis_name="s")

    @functools.partial(
        pl.kernel,
        out_type=[
            jax.ShapeDtypeStruct((NC, Npad, D), jnp.float32),
            jax.ShapeDtypeStruct((NC, Npad, 8), jnp.float32),
        ],
        mesh=mesh,
        compiler_params=pltpu.CompilerParams(use_tc_tiling_on_sc=False),
        scratch_types=[
            pltpu.VMEM_SHARED((Npad, D), jnp.float32),
            pltpu.VMEM_SHARED((Npad, 8), jnp.float32),
            pltpu.VMEM((3, GK * C), jnp.int32),
            pltpu.VMEM((3, GK * C), jnp.int32),
            pltpu.VMEM((2, C), jnp.int32),
            pltpu.VMEM((2, C, D), jnp.float32),
            pltpu.VMEM((C, 8), jnp.float32),
            pltpu.VMEM((ZR, 8), jnp.float32),
            pltpu.SemaphoreType.DMA((2,)),
            pltpu.SemaphoreType.DMA((3,)),
            pltpu.SemaphoreType.DMA((2,)),
        ],
    )
    def sc_agg(feat_hbm, edge_hbm, const_hbm, sum_hbm, cnt_hbm,
               acc, cnt, grow_v, gcol_v, rowm_v, rows_v, ones_v, zc_v,
               sem, gsem, ssem):
        c = lax.axis_index("c")
        s = lax.axis_index("s")
        wid = c * NS + s
        cstart = base_ch * wid

        def load_group(g):
            gb = lax.rem(g, 3)
            off = (cstart + g * GK) * C
            pltpu.async_copy(edge_hbm.at[pl.ds(off, GK * C)],
                             grow_v.at[gb], gsem.at[gb])
            pltpu.async_copy(edge_hbm.at[pl.ds(E + off, GK * C)],
                             gcol_v.at[gb], gsem.at[gb])

        load_group(0)

        zero16 = jnp.zeros((16,), jnp.float32)
        pltpu.async_copy(const_hbm.at[pl.ds(0, C)], ones_v, sem.at[0])
        pltpu.async_copy(const_hbm.at[pl.ds(C, ZR)], zc_v, sem.at[1])

        def fill_z(i, _):
            rows_v[0, i // (D // 16), pl.ds((i % (D // 16)) * 16, 16)] = zero16
            return 0
        lax.fori_loop(0, C * (D // 16), fill_z, 0)

        pltpu.make_async_copy(const_hbm.at[pl.ds(0, C)], ones_v,
                              sem.at[0]).wait()
        pltpu.make_async_copy(const_hbm.at[pl.ds(C, ZR)], zc_v,
                              sem.at[1]).wait()

        rbase = s * rpt
        nz = rpt // C
        rem_rows = rpt - nz * C
        for k in range(nz):
            pltpu.async_copy(rows_v.at[0], acc.at[pl.ds(rbase + k * C, C)],
                             ssem.at[0])
        if rem_rows:
            pltpu.async_copy(rows_v.at[0, pl.ds(0, rem_rows)],
                             acc.at[pl.ds(rbase + nz * C, rem_rows)], ssem.at[0])
        for k in range(8):
            pltpu.async_copy(zc_v, cnt.at[pl.ds(rbase + k * ZR, ZR)], ssem.at[1])
        for k in range(nz):
            pltpu.make_async_copy(rows_v.at[0], acc.at[pl.ds(rbase + k * C, C)],
                                  ssem.at[0]).wait()
        if rem_rows:
            pltpu.make_async_copy(rows_v.at[0, pl.ds(0, rem_rows)],
                                  acc.at[pl.ds(rbase + nz * C, rem_rows)],
                                  ssem.at[0]).wait()
        for k in range(8):
            pltpu.make_async_copy(zc_v, cnt.at[pl.ds(rbase + k * ZR, ZR)],
                                  ssem.at[1]).wait()
        plsc.subcore_barrier()

        def wait_group(g):
            gb = lax.rem(g, 3)
            pltpu.make_async_copy(edge_hbm.at[pl.ds(0, GK * C)],
                                  grow_v.at[gb], gsem.at[gb]).wait()
            pltpu.make_async_copy(edge_hbm.at[pl.ds(0, GK * C)],
                                  gcol_v.at[gb], gsem.at[gb]).wait()

        def stage(i):
            b = lax.rem(i, 2)
            g = i // GK
            gb = lax.rem(g, 3)
            kofs = lax.rem(i, GK) * C
            for j in range(C // 16):
                r = grow_v[gb, pl.ds(kofs + j * 16, 16)]
                cc = gcol_v[gb, pl.ds(kofs + j * 16, 16)]
                rowm_v[b, pl.ds(j * 16, 16)] = jnp.where(r == cc, N, r)
            pltpu.async_copy(feat_hbm.at[gcol_v.at[gb, pl.ds(kofs, C)]],
                             rows_v.at[b], sem.at[b])

        def wait_gather(i):
            b = lax.rem(i, 2)
            g = i // GK
            gb = lax.rem(g, 3)
            kofs = lax.rem(i, GK) * C
            pltpu.make_async_copy(
                feat_hbm.at[gcol_v.at[gb, pl.ds(kofs, C)]],
                rows_v.at[b], sem.at[b]).wait()

        def start_scatters(i):
            b = lax.rem(i, 2)
            pltpu.async_copy(rows_v.at[b], acc.at[rowm_v.at[b]],
                             ssem.at[b], add=True)
            pltpu.async_copy(ones_v, cnt.at[rowm_v.at[b]],
                             ssem.at[b], add=True)

        def wait_scatters(i):
            b = lax.rem(i, 2)
            pltpu.make_async_copy(rows_v.at[b], acc.at[pl.ds(0, C)],
                                  ssem.at[b]).wait()
            pltpu.make_async_copy(ones_v, cnt.at[pl.ds(0, C)],
                                  ssem.at[b]).wait()

        ntot = ngr * GK

        def body(i, _):
            g = i // GK
            at_group = lax.rem(i, GK) == 0

            @pl.when(i > 1)
            def _():
                wait_scatters(i - 2)

            @pl.when(at_group)
            def _():
                wait_group(g)

            @pl.when(jnp.logical_and(at_group, g + 1 < ngr))
            def _():
                load_group(g + 1)

            stage(i)

            @pl.when(i > 0)
            def _():
                wait_gather(i - 1)
                start_scatters(i - 1)
            return 0
        lax.fori_loop(0, ntot, body, 0)
        wait_gather(ntot - 1)
        start_scatters(ntot - 1)
        wait_scatters(ntot - 2)
        wait_scatters(ntot - 1)

        @pl.when(wid < extra)
        def _():
            toff = (NW * base_ch + wid) * C
            pltpu.sync_copy(edge_hbm.at[pl.ds(toff, C)],
                            grow_v.at[0, pl.ds(0, C)])
            pltpu.sync_copy(edge_hbm.at[pl.ds(E + toff, C)],
                            gcol_v.at[0, pl.ds(0, C)])
            for j in range(C // 16):
                r = grow_v[0, pl.ds(j * 16, 16)]
                cc = gcol_v[0, pl.ds(j * 16, 16)]
                rowm_v[0, pl.ds(j * 16, 16)] = jnp.where(r == cc, N, r)
            pltpu.async_copy(feat_hbm.at[gcol_v.at[0, pl.ds(0, C)]],
                             rows_v.at[0], sem.at[0])
            pltpu.make_async_copy(
                feat_hbm.at[gcol_v.at[0, pl.ds(0, C)]],
                rows_v.at[0], sem.at[0]).wait()
            pltpu.sync_copy(rows_v.at[0], acc.at[rowm_v.at[0]], add=True)
            pltpu.sync_copy(ones_v, cnt.at[rowm_v.at[0]], add=True)

        plsc.subcore_barrier()
        pltpu.async_copy(acc.at[pl.ds(rbase, rpt)],
                         sum_hbm.at[c, pl.ds(rbase, rpt)], sem.at[0])
        pltpu.async_copy(cnt.at[pl.ds(rbase, rpt)],
                         cnt_hbm.at[c, pl.ds(rbase, rpt)], sem.at[1])
        pltpu.make_async_copy(acc.at[pl.ds(rbase, rpt)],
                              sum_hbm.at[c, pl.ds(rbase, rpt)], sem.at[0]).wait()
        pltpu.make_async_copy(cnt.at[pl.ds(rbase, rpt)],
                              cnt_hbm.at[c, pl.ds(rbase, rpt)], sem.at[1]).wait()

    return sc_agg


def _div_body(ps_ref, pc_ref, feat_ref, o_ref):
    total = ps_ref[0] + ps_ref[1] + feat_ref[...]
    den = pc_ref[0, :, 0:1] + pc_ref[1, :, 0:1] + 1.0
    o_ref[...] = total / den


def _combine(psum, pcnt, feat):
    N, D = feat.shape
    BN = 400
    grid = (N // BN,)
    return pl.pallas_call(
        _div_body,
        grid=grid,
        in_specs=[
            pl.BlockSpec((2, BN, D), lambda i: (0, i, 0)),
            pl.BlockSpec((2, BN, 8), lambda i: (0, i, 0)),
            pl.BlockSpec((BN, D), lambda i: (i, 0)),
        ],
        out_specs=pl.BlockSpec((BN, D), lambda i: (i, 0)),
        out_shape=jax.ShapeDtypeStruct((N, D), jnp.float32),
    )(psum, pcnt, feat)


def kernel(x, edge_index, W):
    N, _ = x.shape
    D = W.shape[1]
    E = edge_index.shape[1]

    info = plsc.get_sparse_core_info()
    NS = info.num_subcores
    C = 128
    assert E % C == 0
    rpt = -(-(N + 1) // NS)
    rpt = -(-rpt // 8) * 8
    Npad = rpt * NS

    feat = _relu_matmul(x, W)
    const = jnp.concatenate([jnp.ones((C, 8), jnp.float32),
                             jnp.zeros((rpt // 8, 8), jnp.float32)], axis=0)
    psum, pcnt = _make_sc_aggregate(N, E, D, Npad, C, rpt)(
        feat, edge_index.reshape(-1), const)
    return _combine(psum, pcnt, feat)


# speedup vs baseline: 1.1726x; 1.0731x over previous
import functools

import jax
import jax.numpy as jnp
from jax import lax
from jax.experimental import pallas as pl
from jax.experimental.pallas import tpu as pltpu
from jax.experimental.pallas import tpu_sc as plsc


def _mm_body(x_ref, w_ref, o_ref):
    o_ref[...] = jnp.maximum(
        jnp.dot(x_ref[...], w_ref[...], preferred_element_type=jnp.float32), 0.0)


def _relu_matmul(x, W):
    N, Din = x.shape
    Dout = W.shape[1]
    BN = 2000
    grid = (N // BN,)
    return pl.pallas_call(
        _mm_body,
        grid=grid,
        in_specs=[
            pl.BlockSpec((BN, Din), lambda i: (i, 0)),
            pl.BlockSpec((Din, Dout), lambda i: (0, 0)),
        ],
        out_specs=pl.BlockSpec((BN, Dout), lambda i: (i, 0)),
        out_shape=jax.ShapeDtypeStruct((N, Dout), jnp.float32),
    )(x, W)


def _make_sc_aggregate(N, E, D, Npad, C, rpt):
    info = plsc.get_sparse_core_info()
    NC, NS = info.num_cores, info.num_subcores
    NW = NC * NS
    nch = E // C
    base_ch = nch // NW
    extra = nch - base_ch * NW
    GK = 6
    ngr = base_ch // GK
    assert ngr * GK == base_ch
    ZR = rpt // 8

    mesh = plsc.VectorSubcoreMesh(core_axis_name="c", subcore_axis_name="s")

    @functools.partial(
        pl.kernel,
        out_type=[
            jax.ShapeDtypeStruct((NC, Npad, D), jnp.float32),
            jax.ShapeDtypeStruct((NC, Npad, 8), jnp.float32),
        ],
        mesh=mesh,
        compiler_params=pltpu.CompilerParams(use_tc_tiling_on_sc=False),
        scratch_types=[
            pltpu.VMEM_SHARED((Npad, D), jnp.float32),
            pltpu.VMEM_SHARED((Npad, 8), jnp.float32),
            pltpu.VMEM((3, GK * C), jnp.int32),
            pltpu.VMEM((3, GK * C), jnp.int32),
            pltpu.VMEM((2, C), jnp.int32),
            pltpu.VMEM((2, C, D), jnp.float32),
            pltpu.VMEM((C, 8), jnp.float32),
            pltpu.VMEM((ZR, 8), jnp.float32),
            pltpu.SemaphoreType.DMA((2,)),
            pltpu.SemaphoreType.DMA((3,)),
            pltpu.SemaphoreType.DMA((2,)),
        ],
    )
    def sc_agg(feat_hbm, edge_hbm, const_hbm, sum_hbm, cnt_hbm,
               acc, cnt, grow_v, gcol_v, rowm_v, rows_v, ones_v, zc_v,
               sem, gsem, ssem):
        c = lax.axis_index("c")
        s = lax.axis_index("s")
        wid = c * NS + s
        cstart = base_ch * wid

        def load_group(g):
            gb = lax.rem(g, 3)
            off = (cstart + g * GK) * C
            pltpu.async_copy(edge_hbm.at[pl.ds(off, GK * C)],
                             grow_v.at[gb], gsem.at[gb])
            pltpu.async_copy(edge_hbm.at[pl.ds(E + off, GK * C)],
                             gcol_v.at[gb], gsem.at[gb])

        load_group(0)

        zero16 = jnp.zeros((16,), jnp.float32)
        pltpu.async_copy(const_hbm.at[pl.ds(0, C)], ones_v, sem.at[0])
        pltpu.async_copy(const_hbm.at[pl.ds(C, ZR)], zc_v, sem.at[1])

        def fill_z(i, _):
            rows_v[0, i // (D // 16), pl.ds((i % (D // 16)) * 16, 16)] = zero16
            return 0
        lax.fori_loop(0, C * (D // 16), fill_z, 0)

        pltpu.make_async_copy(const_hbm.at[pl.ds(0, C)], ones_v,
                              sem.at[0]).wait()
        pltpu.make_async_copy(const_hbm.at[pl.ds(C, ZR)], zc_v,
                              sem.at[1]).wait()

        rbase = s * rpt
        nz = rpt // C
        rem_rows = rpt - nz * C
        for k in range(nz):
            pltpu.async_copy(rows_v.at[0], acc.at[pl.ds(rbase + k * C, C)],
                             ssem.at[0])
        if rem_rows:
            pltpu.async_copy(rows_v.at[0, pl.ds(0, rem_rows)],
                             acc.at[pl.ds(rbase + nz * C, rem_rows)], ssem.at[0])
        for k in range(8):
            pltpu.async_copy(zc_v, cnt.at[pl.ds(rbase + k * ZR, ZR)], ssem.at[1])
        for k in range(nz):
            pltpu.make_async_copy(rows_v.at[0], acc.at[pl.ds(rbase + k * C, C)],
                                  ssem.at[0]).wait()
        if rem_rows:
            pltpu.make_async_copy(rows_v.at[0, pl.ds(0, rem_rows)],
                                  acc.at[pl.ds(rbase + nz * C, rem_rows)],
                                  ssem.at[0]).wait()
        for k in range(8):
            pltpu.make_async_copy(zc_v, cnt.at[pl.ds(rbase + k * ZR, ZR)],
                                  ssem.at[1]).wait()
        plsc.subcore_barrier()

        def wait_group(g):
            gb = lax.rem(g, 3)
            pltpu.make_async_copy(edge_hbm.at[pl.ds(0, GK * C)],
                                  grow_v.at[gb], gsem.at[gb]).wait()
            pltpu.make_async_copy(edge_hbm.at[pl.ds(0, GK * C)],
                                  gcol_v.at[gb], gsem.at[gb]).wait()

        def stage(i):
            b = lax.rem(i, 2)
            g = i // GK
            gb = lax.rem(g, 3)
            kofs = lax.rem(i, GK) * C
            for j in range(C // 16):
                r = grow_v[gb, pl.ds(kofs + j * 16, 16)]
                cc = gcol_v[gb, pl.ds(kofs + j * 16, 16)]
                rowm_v[b, pl.ds(j * 16, 16)] = jnp.where(r == cc, N, r)
            pltpu.async_copy(feat_hbm.at[gcol_v.at[gb, pl.ds(kofs, C)]],
                             rows_v.at[b], sem.at[b])

        def wait_gather(i):
            b = lax.rem(i, 2)
            g = i // GK
            gb = lax.rem(g, 3)
            kofs = lax.rem(i, GK) * C
            pltpu.make_async_copy(
                feat_hbm.at[gcol_v.at[gb, pl.ds(kofs, C)]],
                rows_v.at[b], sem.at[b]).wait()

        def start_scatters(i):
            b = lax.rem(i, 2)
            pltpu.async_copy(rows_v.at[b], acc.at[rowm_v.at[b]],
                             ssem.at[b], add=True)
            pltpu.async_copy(ones_v, cnt.at[rowm_v.at[b]],
                             ssem.at[b], add=True)

        def wait_scatters(i):
            b = lax.rem(i, 2)
            pltpu.make_async_copy(rows_v.at[b], acc.at[pl.ds(0, C)],
                                  ssem.at[b]).wait()
            pltpu.make_async_copy(ones_v, cnt.at[pl.ds(0, C)],
                                  ssem.at[b]).wait()

        ntot = ngr * GK

        def body(i, _):
            g = i // GK
            at_group = lax.rem(i, GK) == 0

            @pl.when(i > 1)
            def _():
                wait_scatters(i - 2)

            @pl.when(at_group)
            def _():
                wait_group(g)

            @pl.when(jnp.logical_and(at_group, g + 1 < ngr))
            def _():
                load_group(g + 1)

            stage(i)

            @pl.when(i > 0)
            def _():
                wait_gather(i - 1)
                start_scatters(i - 1)
            return 0
        lax.fori_loop(0, ntot, body, 0)
        wait_gather(ntot - 1)
        start_scatters(ntot - 1)
        wait_scatters(ntot - 2)
        wait_scatters(ntot - 1)

        @pl.when(wid < extra)
        def _():
            toff = (NW * base_ch + wid) * C
            pltpu.sync_copy(edge_hbm.at[pl.ds(toff, C)],
                            grow_v.at[0, pl.ds(0, C)])
            pltpu.sync_copy(edge_hbm.at[pl.ds(E + toff, C)],
                            gcol_v.at[0, pl.ds(0, C)])
            for j in range(C // 16):
                r = grow_v[0, pl.ds(j * 16, 16)]
                cc = gcol_v[0, pl.ds(j * 16, 16)]
                rowm_v[0, pl.ds(j * 16, 16)] = jnp.where(r == cc, N, r)
            pltpu.async_copy(feat_hbm.at[gcol_v.at[0, pl.ds(0, C)]],
                             rows_v.at[0], sem.at[0])
            pltpu.make_async_copy(
                feat_hbm.at[gcol_v.at[0, pl.ds(0, C)]],
                rows_v.at[0], sem.at[0]).wait()
            pltpu.sync_copy(rows_v.at[0], acc.at[rowm_v.at[0]], add=True)
            pltpu.sync_copy(ones_v, cnt.at[rowm_v.at[0]], add=True)

        plsc.subcore_barrier()
        pltpu.async_copy(acc.at[pl.ds(rbase, rpt)],
                         sum_hbm.at[c, pl.ds(rbase, rpt)], sem.at[0])
        pltpu.async_copy(cnt.at[pl.ds(rbase, rpt)],
                         cnt_hbm.at[c, pl.ds(rbase, rpt)], sem.at[1])
        pltpu.make_async_copy(acc.at[pl.ds(rbase, rpt)],
                              sum_hbm.at[c, pl.ds(rbase, rpt)], sem.at[0]).wait()
        pltpu.make_async_copy(cnt.at[pl.ds(rbase, rpt)],
                              cnt_hbm.at[c, pl.ds(rbase, rpt)], sem.at[1]).wait()

    return sc_agg


def _div_body(ps_ref, pc_ref, feat_ref, o_ref):
    total = ps_ref[0] + ps_ref[1] + feat_ref[...]
    den = pc_ref[0, :, 0:1] + pc_ref[1, :, 0:1] + 1.0
    o_ref[...] = total / den


def _combine(psum, pcnt, feat):
    N, D = feat.shape
    BN = 2000
    grid = (N // BN,)
    return pl.pallas_call(
        _div_body,
        grid=grid,
        in_specs=[
            pl.BlockSpec((2, BN, D), lambda i: (0, i, 0)),
            pl.BlockSpec((2, BN, 8), lambda i: (0, i, 0)),
            pl.BlockSpec((BN, D), lambda i: (i, 0)),
        ],
        out_specs=pl.BlockSpec((BN, D), lambda i: (i, 0)),
        out_shape=jax.ShapeDtypeStruct((N, D), jnp.float32),
    )(psum, pcnt, feat)


def kernel(x, edge_index, W):
    N, _ = x.shape
    D = W.shape[1]
    E = edge_index.shape[1]

    info = plsc.get_sparse_core_info()
    NS = info.num_subcores
    C = 128
    assert E % C == 0
    rpt = -(-(N + 1) // NS)
    rpt = -(-rpt // 8) * 8
    Npad = rpt * NS

    feat = _relu_matmul(x, W)
    const = jnp.concatenate([jnp.ones((C, 8), jnp.float32),
                             jnp.zeros((rpt // 8, 8), jnp.float32)], axis=0)
    psum, pcnt = _make_sc_aggregate(N, E, D, Npad, C, rpt)(
        feat, edge_index.reshape(-1), const)
    return _combine(psum, pcnt, feat)


# speedup vs baseline: 1.1911x; 1.0157x over previous
import functools

import jax
import jax.numpy as jnp
from jax import lax
from jax.experimental import pallas as pl
from jax.experimental.pallas import tpu as pltpu
from jax.experimental.pallas import tpu_sc as plsc


def _mm_body(x_ref, w_ref, o_ref):
    o_ref[...] = jnp.maximum(
        jnp.dot(x_ref[...], w_ref[...], preferred_element_type=jnp.float32), 0.0)


def _relu_matmul(x, W):
    N, Din = x.shape
    Dout = W.shape[1]
    BN = 5000
    grid = (N // BN,)
    return pl.pallas_call(
        _mm_body,
        grid=grid,
        in_specs=[
            pl.BlockSpec((BN, Din), lambda i: (i, 0)),
            pl.BlockSpec((Din, Dout), lambda i: (0, 0)),
        ],
        out_specs=pl.BlockSpec((BN, Dout), lambda i: (i, 0)),
        out_shape=jax.ShapeDtypeStruct((N, Dout), jnp.float32),
    )(x, W)


def _make_sc_aggregate(N, E, D, Npad, C, rpt):
    info = plsc.get_sparse_core_info()
    NC, NS = info.num_cores, info.num_subcores
    NW = NC * NS
    nch = E // C
    base_ch = nch // NW
    extra = nch - base_ch * NW
    GK = 6
    ngr = base_ch // GK
    assert ngr * GK == base_ch
    ZR = rpt // 8

    mesh = plsc.VectorSubcoreMesh(core_axis_name="c", subcore_axis_name="s")

    @functools.partial(
        pl.kernel,
        out_type=[
            jax.ShapeDtypeStruct((NC, Npad, D), jnp.float32),
            jax.ShapeDtypeStruct((NC, Npad, 8), jnp.float32),
        ],
        mesh=mesh,
        compiler_params=pltpu.CompilerParams(use_tc_tiling_on_sc=False),
        scratch_types=[
            pltpu.VMEM_SHARED((Npad, D), jnp.float32),
            pltpu.VMEM_SHARED((Npad, 8), jnp.float32),
            pltpu.VMEM((3, GK * C), jnp.int32),
            pltpu.VMEM((3, GK * C), jnp.int32),
            pltpu.VMEM((2, C), jnp.int32),
            pltpu.VMEM((2, C, D), jnp.float32),
            pltpu.VMEM((C, 8), jnp.float32),
            pltpu.VMEM((ZR, 8), jnp.float32),
            pltpu.SemaphoreType.DMA((2,)),
            pltpu.SemaphoreType.DMA((3,)),
            pltpu.SemaphoreType.DMA((2,)),
        ],
    )
    def sc_agg(feat_hbm, edge_hbm, const_hbm, sum_hbm, cnt_hbm,
               acc, cnt, grow_v, gcol_v, rowm_v, rows_v, ones_v, zc_v,
               sem, gsem, ssem):
        c = lax.axis_index("c")
        s = lax.axis_index("s")
        wid = c * NS + s
        cstart = base_ch * wid

        def load_group(g):
            gb = lax.rem(g, 3)
            off = (cstart + g * GK) * C
            pltpu.async_copy(edge_hbm.at[pl.ds(off, GK * C)],
                             grow_v.at[gb], gsem.at[gb])
            pltpu.async_copy(edge_hbm.at[pl.ds(E + off, GK * C)],
                             gcol_v.at[gb], gsem.at[gb])

        load_group(0)

        zero16 = jnp.zeros((16,), jnp.float32)
        pltpu.async_copy(const_hbm.at[pl.ds(0, C)], ones_v, sem.at[0])
        pltpu.async_copy(const_hbm.at[pl.ds(C, ZR)], zc_v, sem.at[1])

        def fill_z(i, _):
            rows_v[0, i // (D // 16), pl.ds((i % (D // 16)) * 16, 16)] = zero16
            return 0
        lax.fori_loop(0, C * (D // 16), fill_z, 0)

        pltpu.make_async_copy(const_hbm.at[pl.ds(0, C)], ones_v,
                              sem.at[0]).wait()
        pltpu.make_async_copy(const_hbm.at[pl.ds(C, ZR)], zc_v,
                              sem.at[1]).wait()

        rbase = s * rpt
        nz = rpt // C
        rem_rows = rpt - nz * C
        for k in range(nz):
            pltpu.async_copy(rows_v.at[0], acc.at[pl.ds(rbase + k * C, C)],
                             ssem.at[0])
        if rem_rows:
            pltpu.async_copy(rows_v.at[0, pl.ds(0, rem_rows)],
                             acc.at[pl.ds(rbase + nz * C, rem_rows)], ssem.at[0])
        for k in range(8):
            pltpu.async_copy(zc_v, cnt.at[pl.ds(rbase + k * ZR, ZR)], ssem.at[1])
        for k in range(nz):
            pltpu.make_async_copy(rows_v.at[0], acc.at[pl.ds(rbase + k * C, C)],
                                  ssem.at[0]).wait()
        if rem_rows:
            pltpu.make_async_copy(rows_v.at[0, pl.ds(0, rem_rows)],
                                  acc.at[pl.ds(rbase + nz * C, rem_rows)],
                                  ssem.at[0]).wait()
        for k in range(8):
            pltpu.make_async_copy(zc_v, cnt.at[pl.ds(rbase + k * ZR, ZR)],
                                  ssem.at[1]).wait()
        plsc.subcore_barrier()

        def wait_group(g):
            gb = lax.rem(g, 3)
            pltpu.make_async_copy(edge_hbm.at[pl.ds(0, GK * C)],
                                  grow_v.at[gb], gsem.at[gb]).wait()
            pltpu.make_async_copy(edge_hbm.at[pl.ds(0, GK * C)],
                                  gcol_v.at[gb], gsem.at[gb]).wait()

        def stage(i):
            b = lax.rem(i, 2)
            g = i // GK
            gb = lax.rem(g, 3)
            kofs = lax.rem(i, GK) * C
            for j in range(C // 16):
                r = grow_v[gb, pl.ds(kofs + j * 16, 16)]
                cc = gcol_v[gb, pl.ds(kofs + j * 16, 16)]
                rowm_v[b, pl.ds(j * 16, 16)] = jnp.where(r == cc, N, r)
            pltpu.async_copy(feat_hbm.at[gcol_v.at[gb, pl.ds(kofs, C)]],
                             rows_v.at[b], sem.at[b])

        def wait_gather(i):
            b = lax.rem(i, 2)
            g = i // GK
            gb = lax.rem(g, 3)
            kofs = lax.rem(i, GK) * C
            pltpu.make_async_copy(
                feat_hbm.at[gcol_v.at[gb, pl.ds(kofs, C)]],
                rows_v.at[b], sem.at[b]).wait()

        def start_scatters(i):
            b = lax.rem(i, 2)
            pltpu.async_copy(rows_v.at[b], acc.at[rowm_v.at[b]],
                             ssem.at[b], add=True)
            pltpu.async_copy(ones_v, cnt.at[rowm_v.at[b]],
                             ssem.at[b], add=True)

        def wait_scatters(i):
            b = lax.rem(i, 2)
            pltpu.make_async_copy(rows_v.at[b], acc.at[pl.ds(0, C)],
                                  ssem.at[b]).wait()
            pltpu.make_async_copy(ones_v, cnt.at[pl.ds(0, C)],
                                  ssem.at[b]).wait()

        ntot = ngr * GK

        def body(i, _):
            g = i // GK
            at_group = lax.rem(i, GK) == 0

            @pl.when(i > 1)
            def _():
                wait_scatters(i - 2)

            @pl.when(at_group)
            def _():
                wait_group(g)

            @pl.when(jnp.logical_and(at_group, g + 1 < ngr))
            def _():
                load_group(g + 1)

            stage(i)

            @pl.when(i > 0)
            def _():
                wait_gather(i - 1)
                start_scatters(i - 1)
            return 0
        lax.fori_loop(0, ntot, body, 0)
        wait_gather(ntot - 1)
        start_scatters(ntot - 1)
        wait_scatters(ntot - 2)
        wait_scatters(ntot - 1)

        @pl.when(wid < extra)
        def _():
            toff = (NW * base_ch + wid) * C
            pltpu.sync_copy(edge_hbm.at[pl.ds(toff, C)],
                            grow_v.at[0, pl.ds(0, C)])
            pltpu.sync_copy(edge_hbm.at[pl.ds(E + toff, C)],
                            gcol_v.at[0, pl.ds(0, C)])
            for j in range(C // 16):
                r = grow_v[0, pl.ds(j * 16, 16)]
                cc = gcol_v[0, pl.ds(j * 16, 16)]
                rowm_v[0, pl.ds(j * 16, 16)] = jnp.where(r == cc, N, r)
            pltpu.async_copy(feat_hbm.at[gcol_v.at[0, pl.ds(0, C)]],
                             rows_v.at[0], sem.at[0])
            pltpu.make_async_copy(
                feat_hbm.at[gcol_v.at[0, pl.ds(0, C)]],
                rows_v.at[0], sem.at[0]).wait()
            pltpu.sync_copy(rows_v.at[0], acc.at[rowm_v.at[0]], add=True)
            pltpu.sync_copy(ones_v, cnt.at[rowm_v.at[0]], add=True)

        plsc.subcore_barrier()
        pltpu.async_copy(acc.at[pl.ds(rbase, rpt)],
                         sum_hbm.at[c, pl.ds(rbase, rpt)], sem.at[0])
        pltpu.async_copy(cnt.at[pl.ds(rbase, rpt)],
                         cnt_hbm.at[c, pl.ds(rbase, rpt)], sem.at[1])
        pltpu.make_async_copy(acc.at[pl.ds(rbase, rpt)],
                              sum_hbm.at[c, pl.ds(rbase, rpt)], sem.at[0]).wait()
        pltpu.make_async_copy(cnt.at[pl.ds(rbase, rpt)],
                              cnt_hbm.at[c, pl.ds(rbase, rpt)], sem.at[1]).wait()

    return sc_agg


def _div_body(ps_ref, pc_ref, feat_ref, o_ref):
    total = ps_ref[0] + ps_ref[1] + feat_ref[...]
    den = pc_ref[0, :, 0:1] + pc_ref[1, :, 0:1] + 1.0
    o_ref[...] = total / den


def _combine(psum, pcnt, feat):
    N, D = feat.shape
    BN = 5000
    grid = (N // BN,)
    return pl.pallas_call(
        _div_body,
        grid=grid,
        in_specs=[
            pl.BlockSpec((2, BN, D), lambda i: (0, i, 0)),
            pl.BlockSpec((2, BN, 8), lambda i: (0, i, 0)),
            pl.BlockSpec((BN, D), lambda i: (i, 0)),
        ],
        out_specs=pl.BlockSpec((BN, D), lambda i: (i, 0)),
        out_shape=jax.ShapeDtypeStruct((N, D), jnp.float32),
    )(psum, pcnt, feat)


def kernel(x, edge_index, W):
    N, _ = x.shape
    D = W.shape[1]
    E = edge_index.shape[1]

    info = plsc.get_sparse_core_info()
    NS = info.num_subcores
    C = 128
    assert E % C == 0
    rpt = -(-(N + 1) // NS)
    rpt = -(-rpt // 8) * 8
    Npad = rpt * NS

    feat = _relu_matmul(x, W)
    const = jnp.concatenate([jnp.ones((C, 8), jnp.float32),
                             jnp.zeros((rpt // 8, 8), jnp.float32)], axis=0)
    psum, pcnt = _make_sc_aggregate(N, E, D, Npad, C, rpt)(
        feat, edge_index.reshape(-1), const)
    return _combine(psum, pcnt, feat)
